# baseline (speedup 1.0000x reference)
import functools

import jax
import jax.numpy as jnp
from jax import lax
from jax.experimental import pallas as pl
from jax.experimental.pallas import tpu as pltpu
from jax.experimental.pallas import tpu_sc as plsc

NUM_CODES = 1024
DIM = 64
COMMIT = 0.25
K = 3
ROWS = 16384
BLOCK = 1024
GRID = ROWS // BLOCK

_NC, _NS = 2, 16
_NW = _NC * _NS
_RPW = ROWS // _NW
_CHUNK = 128
_NCHUNK = _RPW // _CHUNK


def _tc_body(x_ref, e_ref, enc_ref, i0_ref, i1_ref, i2_ref, loss_ref,
             perp_ref, cnt_ref, acc_ref):
    i = pl.program_id(0)
    x = x_ref[...]
    e = e_ref[...]

    xsq = jnp.sum(x * x, axis=1, keepdims=True)
    esq = jnp.sum(e * e, axis=1)[None, :]
    xe = jax.lax.dot_general(x, e, (((1,), (1,)), ((), ())),
                             preferred_element_type=jnp.float32)
    d = xsq + esq - 2.0 * xe

    iota = jax.lax.broadcasted_iota(jnp.int32, d.shape, 1)
    idx_refs = (i0_ref, i1_ref, i2_ref)
    for k in range(K):
        dmin = jnp.min(d, axis=1, keepdims=True)
        idx = jnp.min(jnp.where(d == dmin, iota, NUM_CODES), axis=1,
                      keepdims=True)
        idx_refs[k][...] = idx
        if k == 0:
            part_loss = jnp.sum(dmin, keepdims=True)[:1, :1]
        if k < K - 1:
            d = jnp.where(iota == idx, jnp.inf, d)

    last_oh = (iota == idx).astype(jnp.float32)
    enc_ref[...] = last_oh

    @pl.when(i == 0)
    def _():
        acc_ref[...] = jnp.zeros_like(acc_ref)
        cnt_ref[...] = jnp.zeros_like(cnt_ref)

    acc_ref[...] += part_loss
    cnt_ref[...] += jnp.sum(last_oh, axis=0, keepdims=True)

    @pl.when(i == GRID - 1)
    def _():
        loss_ref[...] = acc_ref[...] * (COMMIT / (ROWS * DIM))
        p = cnt_ref[...] * (1.0 / ROWS)
        perp_ref[...] = jnp.exp(-jnp.sum(p * jnp.log(p + 1e-10),
                                         keepdims=True))


@functools.cache
def _sc_gather_fn():
    mesh = plsc.VectorSubcoreMesh(core_axis_name="c", subcore_axis_name="s")
    row_ty = jax.ShapeDtypeStruct((ROWS, DIM), jnp.float32)

    @functools.partial(
        pl.kernel, mesh=mesh,
        compiler_params=pltpu.CompilerParams(use_tc_tiling_on_sc=False),
        out_type=(row_ty, row_ty, row_ty),
        scratch_types=[
            pltpu.VMEM((K * _RPW,), jnp.int32),
            pltpu.VMEM((K * _RPW, DIM), jnp.float32),
            pltpu.SemaphoreType.DMA,
            pltpu.SemaphoreType.DMA,
        ],
    )
    def _sc_gather(table_hbm, i0_hbm, i1_hbm, i2_hbm, o0_hbm, o1_hbm, o2_hbm,
                   idx_v, rows_v, gsem, ssem):
        wid = lax.axis_index("s") * _NC + lax.axis_index("c")
        base = wid * _RPW
        for k, idx_hbm in enumerate((i0_hbm, i1_hbm, i2_hbm)):
            pltpu.sync_copy(idx_hbm.at[pl.ds(base, _RPW)],
                            idx_v.at[pl.ds(k * _RPW, _RPW)])
        copies = []
        for c in range(K * _NCHUNK):
            copies.append(pltpu.async_copy(
                table_hbm.at[idx_v.at[pl.ds(c * _CHUNK, _CHUNK)]],
                rows_v.at[pl.ds(c * _CHUNK, _CHUNK)], gsem))
        stores = []
        for k, out_hbm in enumerate((o0_hbm, o1_hbm, o2_hbm)):
            for cp in copies[k * _NCHUNK:(k + 1) * _NCHUNK]:
                cp.wait()
            stores.append(pltpu.async_copy(
                rows_v.at[pl.ds(k * _RPW, _RPW)],
                out_hbm.at[pl.ds(base, _RPW)], ssem))
        for st in stores:
            st.wait()

    return _sc_gather


def kernel(inputs, embedding_weight):
    flat = jnp.transpose(inputs, (0, 2, 3, 1)).reshape(ROWS, DIM)

    out_shapes = (
        jax.ShapeDtypeStruct((ROWS, NUM_CODES), jnp.float32),
        jax.ShapeDtypeStruct((ROWS, 1), jnp.int32),
        jax.ShapeDtypeStruct((ROWS, 1), jnp.int32),
        jax.ShapeDtypeStruct((ROWS, 1), jnp.int32),
        jax.ShapeDtypeStruct((1, 1), jnp.float32),
        jax.ShapeDtypeStruct((1, 1), jnp.float32),
    )
    enc, i0, i1, i2, loss, perp = pl.pallas_call(
        _tc_body,
        grid=(GRID,),
        in_specs=[
            pl.BlockSpec((BLOCK, DIM), lambda i: (i, 0)),
            pl.BlockSpec((NUM_CODES, DIM), lambda i: (0, 0)),
        ],
        out_specs=[
            pl.BlockSpec((BLOCK, NUM_CODES), lambda i: (i, 0)),
            pl.BlockSpec((BLOCK, 1), lambda i: (i, 0)),
            pl.BlockSpec((BLOCK, 1), lambda i: (i, 0)),
            pl.BlockSpec((BLOCK, 1), lambda i: (i, 0)),
            pl.BlockSpec((1, 1), lambda i: (0, 0)),
            pl.BlockSpec((1, 1), lambda i: (0, 0)),
        ],
        scratch_shapes=[
            pltpu.VMEM((1, NUM_CODES), jnp.float32),
            pltpu.VMEM((1, 1), jnp.float32),
        ],
        out_shape=out_shapes,
    )(flat, embedding_weight)

    q0, q1, q2 = _sc_gather_fn()(embedding_weight, i0.reshape(ROWS),
                                 i1.reshape(ROWS), i2.reshape(ROWS))
    in_shape = (16, 32, 32, DIM)
    q0r = q0.reshape(in_shape)
    q1r = q1.reshape(in_shape)
    q2r = q2.reshape(in_shape)
    quantized = jnp.transpose(q0r, (0, 3, 1, 2))
    return (loss[0, 0], quantized, perp[0, 0], enc, (q0r, q1r, q2r))


# speedup vs baseline: 1.0288x; 1.0288x over previous
import functools

import jax
import jax.numpy as jnp
from jax import lax
from jax.experimental import pallas as pl
from jax.experimental.pallas import tpu as pltpu
from jax.experimental.pallas import tpu_sc as plsc

NUM_CODES = 1024
DIM = 64
COMMIT = 0.25
K = 3
ROWS = 16384
BLOCK = 1024
GRID = ROWS // BLOCK

_NC, _NS = 2, 16
_NW = _NC * _NS
_RPW = ROWS // _NW
_CHUNK = 256
_NCHUNK = _RPW // _CHUNK


def _tc_body(x_ref, e_ref, enc_ref, i0_ref, i1_ref, i2_ref, loss_ref,
             perp_ref, cnt_ref, acc_ref):
    i = pl.program_id(0)
    x = x_ref[...]
    e = e_ref[...]

    xsq = jnp.sum(x * x, axis=1, keepdims=True)
    esq = jnp.sum(e * e, axis=1)[None, :]
    xe = jax.lax.dot_general(x, e, (((1,), (1,)), ((), ())),
                             preferred_element_type=jnp.float32)
    d = xsq + esq - 2.0 * xe

    iota = jax.lax.broadcasted_iota(jnp.int32, d.shape, 1)
    idx_refs = (i0_ref, i1_ref, i2_ref)
    for k in range(K):
        dmin = jnp.min(d, axis=1, keepdims=True)
        idx = jnp.min(jnp.where(d == dmin, iota, NUM_CODES), axis=1,
                      keepdims=True)
        idx_refs[k][...] = idx
        if k == 0:
            part_loss = jnp.sum(dmin, keepdims=True)[:1, :1]
        if k < K - 1:
            d = jnp.where(iota == idx, jnp.inf, d)

    last_oh = (iota == idx).astype(jnp.float32)
    enc_ref[...] = last_oh

    @pl.when(i == 0)
    def _():
        acc_ref[...] = jnp.zeros_like(acc_ref)
        cnt_ref[...] = jnp.zeros_like(cnt_ref)

    acc_ref[...] += part_loss
    cnt_ref[...] += jnp.sum(last_oh, axis=0, keepdims=True)

    @pl.when(i == GRID - 1)
    def _():
        loss_ref[...] = acc_ref[...] * (COMMIT / (ROWS * DIM))
        p = cnt_ref[...] * (1.0 / ROWS)
        perp_ref[...] = jnp.exp(-jnp.sum(p * jnp.log(p + 1e-10),
                                         keepdims=True))


@functools.cache
def _sc_gather_fn():
    mesh = plsc.VectorSubcoreMesh(core_axis_name="c", subcore_axis_name="s")
    row_ty = jax.ShapeDtypeStruct((ROWS, DIM), jnp.float32)

    @functools.partial(
        pl.kernel, mesh=mesh,
        compiler_params=pltpu.CompilerParams(use_tc_tiling_on_sc=False),
        out_type=(row_ty, row_ty, row_ty),
        scratch_types=[
            pltpu.VMEM((K * _RPW,), jnp.int32),
            pltpu.VMEM((K * _RPW, DIM), jnp.float32),
            pltpu.SemaphoreType.DMA,
        ],
    )
    def _sc_gather(table_hbm, i0_hbm, i1_hbm, i2_hbm, o0_hbm, o1_hbm, o2_hbm,
                   idx_v, rows_v, sem):
        wid = lax.axis_index("s") * _NC + lax.axis_index("c")
        base = wid * _RPW
        for k, idx_hbm in enumerate((i0_hbm, i1_hbm, i2_hbm)):
            pltpu.sync_copy(idx_hbm.at[pl.ds(base, _RPW)],
                            idx_v.at[pl.ds(k * _RPW, _RPW)])
        copies = []
        for c in range(K * _NCHUNK):
            copies.append(pltpu.async_copy(
                table_hbm.at[idx_v.at[pl.ds(c * _CHUNK, _CHUNK)]],
                rows_v.at[pl.ds(c * _CHUNK, _CHUNK)], sem))
        for cp in copies:
            cp.wait()
        for k, out_hbm in enumerate((o0_hbm, o1_hbm, o2_hbm)):
            pltpu.sync_copy(rows_v.at[pl.ds(k * _RPW, _RPW)],
                            out_hbm.at[pl.ds(base, _RPW)])

    return _sc_gather


def kernel(inputs, embedding_weight):
    flat = jnp.transpose(inputs, (0, 2, 3, 1)).reshape(ROWS, DIM)

    out_shapes = (
        jax.ShapeDtypeStruct((ROWS, NUM_CODES), jnp.float32),
        jax.ShapeDtypeStruct((ROWS, 1), jnp.int32),
        jax.ShapeDtypeStruct((ROWS, 1), jnp.int32),
        jax.ShapeDtypeStruct((ROWS, 1), jnp.int32),
        jax.ShapeDtypeStruct((1, 1), jnp.float32),
        jax.ShapeDtypeStruct((1, 1), jnp.float32),
    )
    enc, i0, i1, i2, loss, perp = pl.pallas_call(
        _tc_body,
        grid=(GRID,),
        in_specs=[
            pl.BlockSpec((BLOCK, DIM), lambda i: (i, 0)),
            pl.BlockSpec((NUM_CODES, DIM), lambda i: (0, 0)),
        ],
        out_specs=[
            pl.BlockSpec((BLOCK, NUM_CODES), lambda i: (i, 0)),
            pl.BlockSpec((BLOCK, 1), lambda i: (i, 0)),
            pl.BlockSpec((BLOCK, 1), lambda i: (i, 0)),
            pl.BlockSpec((BLOCK, 1), lambda i: (i, 0)),
            pl.BlockSpec((1, 1), lambda i: (0, 0)),
            pl.BlockSpec((1, 1), lambda i: (0, 0)),
        ],
        scratch_shapes=[
            pltpu.VMEM((1, NUM_CODES), jnp.float32),
            pltpu.VMEM((1, 1), jnp.float32),
        ],
        out_shape=out_shapes,
    )(flat, embedding_weight)

    q0, q1, q2 = _sc_gather_fn()(embedding_weight, i0.reshape(ROWS),
                                 i1.reshape(ROWS), i2.reshape(ROWS))
    in_shape = (16, 32, 32, DIM)
    q0r = q0.reshape(in_shape)
    q1r = q1.reshape(in_shape)
    q2r = q2.reshape(in_shape)
    quantized = jnp.transpose(q0r, (0, 3, 1, 2))
    return (loss[0, 0], quantized, perp[0, 0], enc, (q0r, q1r, q2r))
